# baseline (speedup 1.0000x reference)
import jax
import jax.numpy as jnp
from jax import lax
from jax.experimental import pallas as pl
from jax.experimental.pallas import tpu as pltpu
from jax.experimental.pallas import tpu_sc as plsc

N_NODES = 100000
HIDDEN = 128
CHUNK = 128
N_CHUNKS = (N_NODES + CHUNK - 1) // CHUNK
TAIL = N_NODES - (N_CHUNKS - 1) * CHUNK
BIG_W = N_CHUNKS - 24 * 32
KBIG, KSML = 25, 24
VOCAB = 512
LAST_W_ROW0 = (BIG_W * KBIG + (31 - BIG_W) * KSML) * CHUNK
LAST_FULL = (KSML - 1) * CHUNK


def _body(in_idx, out_idx, tin, tout, w_in, w_out, out, idx_a, idx_b, ba0,
          ba1, bb0, bb1, spm_i, spm_o, sa0, sa1, sb0, sb1, sst):
  cid = lax.axis_index("c")
  sid = lax.axis_index("s")
  wid = sid * 2 + cid

  rpt = VOCAB // 16
  ti = pltpu.async_copy(w_in.at[pl.ds(sid * rpt, rpt)],
                        spm_i.at[pl.ds(sid * rpt, rpt)], sst)
  to = pltpu.async_copy(w_out.at[pl.ds(sid * rpt, rpt)],
                        spm_o.at[pl.ds(sid * rpt, rpt)], sst)
  ti.wait()
  to.wait()
  plsc.subcore_barrier()

  bufs = ((ba0, bb0, sa0, sb0), (ba1, bb1, sa1, sb1))

  def run(n_chunks, chunk0):
    def issue(j, slot):
      ba, bb, sa, sb = bufs[slot]
      ia = idx_a.at[pl.ds(j * CHUNK, CHUNK)]
      ib = idx_b.at[pl.ds(j * CHUNK, CHUNK)]
      pltpu.async_copy(spm_i.at[ia], ba, sa)
      pltpu.async_copy(spm_o.at[ib], bb, sb)

    def finish(j, slot):
      ba, bb, sa, sb = bufs[slot]
      ia = idx_a.at[pl.ds(j * CHUNK, CHUNK)]
      ib = idx_b.at[pl.ds(j * CHUNK, CHUNK)]
      pltpu.make_async_copy(spm_i.at[ia], ba, sa).wait()
      pltpu.make_async_copy(spm_o.at[ib], bb, sb).wait()

      @plsc.parallel_loop(0, CHUNK, unroll=8)
      def _(r):
        for k in range(HIDDEN // 16):
          s = pl.ds(k * 16, 16)
          plsc.addupdate(ba.at[r, s], bb[r, s])

      g = chunk0 + j

      @pl.when(g < N_CHUNKS - 1)
      def _():
        pltpu.sync_copy(ba, out.at[pl.ds(g * CHUNK, CHUNK)])

      @pl.when(g == N_CHUNKS - 1)
      def _():
        pltpu.sync_copy(ba.at[pl.ds(0, TAIL)],
                        out.at[pl.ds(g * CHUNK, TAIL)])

    issue(0, 0)

    def pair_body(p, carry):
      for s in range(2):
        j = 2 * p + s

        @pl.when(j + 1 < n_chunks)
        def _():
          issue(j + 1, 1 - s)

        finish(j, s)
      return carry

    lax.fori_loop(0, n_chunks // 2, pair_body, 0)
    if n_chunks % 2:
      finish(n_chunks - 1, (n_chunks - 1) % 2)

  @pl.when(wid < BIG_W)
  def _():
    n = KBIG * CHUNK
    c0 = wid * KBIG
    pltpu.sync_copy(in_idx.at[pl.ds(c0 * CHUNK, n)], idx_a.at[pl.ds(0, n)])
    pltpu.sync_copy(out_idx.at[pl.ds(c0 * CHUNK, n)], idx_b.at[pl.ds(0, n)])
    run(KBIG, c0)

  @pl.when((wid >= BIG_W) & (wid < 31))
  def _():
    n = KSML * CHUNK
    c0 = BIG_W * KBIG + (wid - BIG_W) * KSML
    pltpu.sync_copy(in_idx.at[pl.ds(c0 * CHUNK, n)], idx_a.at[pl.ds(0, n)])
    pltpu.sync_copy(out_idx.at[pl.ds(c0 * CHUNK, n)], idx_b.at[pl.ds(0, n)])
    run(KSML, c0)

  @pl.when(wid == 31)
  def _():
    pltpu.sync_copy(in_idx.at[pl.ds(LAST_W_ROW0, LAST_FULL)],
                    idx_a.at[pl.ds(0, LAST_FULL)])
    pltpu.sync_copy(out_idx.at[pl.ds(LAST_W_ROW0, LAST_FULL)],
                    idx_b.at[pl.ds(0, LAST_FULL)])
    pltpu.sync_copy(tin, idx_a.at[pl.ds(LAST_FULL, CHUNK)])
    pltpu.sync_copy(tout, idx_b.at[pl.ds(LAST_FULL, CHUNK)])
    run(KSML, LAST_W_ROW0 // CHUNK)


@jax.jit
def kernel(in_deg, out_deg, W_in, W_out):
  in_p = in_deg.astype(jnp.int32)
  out_p = out_deg.astype(jnp.int32)
  tin = jnp.pad(in_p[(N_CHUNKS - 1) * CHUNK:], (0, CHUNK - TAIL))
  tout = jnp.pad(out_p[(N_CHUNKS - 1) * CHUNK:], (0, CHUNK - TAIL))

  mesh = plsc.VectorSubcoreMesh(core_axis_name="c", subcore_axis_name="s")
  f = pl.kernel(
      _body,
      out_type=jax.ShapeDtypeStruct((N_NODES, HIDDEN), jnp.float32),
      mesh=mesh,
      scratch_types=[
          pltpu.VMEM((KBIG * CHUNK,), jnp.int32),
          pltpu.VMEM((KBIG * CHUNK,), jnp.int32),
          pltpu.VMEM((CHUNK, HIDDEN), jnp.float32),
          pltpu.VMEM((CHUNK, HIDDEN), jnp.float32),
          pltpu.VMEM((CHUNK, HIDDEN), jnp.float32),
          pltpu.VMEM((CHUNK, HIDDEN), jnp.float32),
          pltpu.VMEM_SHARED((VOCAB, HIDDEN), jnp.float32),
          pltpu.VMEM_SHARED((VOCAB, HIDDEN), jnp.float32),
          pltpu.SemaphoreType.DMA,
          pltpu.SemaphoreType.DMA,
          pltpu.SemaphoreType.DMA,
          pltpu.SemaphoreType.DMA,
          pltpu.SemaphoreType.DMA,
      ],
  )
  return f(in_p, out_p, tin, tout, W_in, W_out)


# speedup vs baseline: 1.0185x; 1.0185x over previous
import jax
import jax.numpy as jnp
from jax import lax
from jax.experimental import pallas as pl
from jax.experimental.pallas import tpu as pltpu
from jax.experimental.pallas import tpu_sc as plsc

N_NODES = 100000
HIDDEN = 128
CHUNK = 128
N_CHUNKS = (N_NODES + CHUNK - 1) // CHUNK
TAIL = N_NODES - (N_CHUNKS - 1) * CHUNK
BIG_W = N_CHUNKS - 24 * 32
KBIG, KSML = 25, 24
VOCAB = 512
LAST_W_ROW0 = (BIG_W * KBIG + (31 - BIG_W) * KSML) * CHUNK
LAST_FULL = (KSML - 1) * CHUNK


def _body(in_idx, out_idx, tin, tout, w_in, w_out, out, idx_a, idx_b, ba0,
          ba1, bb0, bb1, spm_i, spm_o, sa0, sa1, sb0, sb1, sst):
  cid = lax.axis_index("c")
  sid = lax.axis_index("s")
  wid = sid * 2 + cid

  rpt = VOCAB // 16
  ti = pltpu.async_copy(w_in.at[pl.ds(sid * rpt, rpt)],
                        spm_i.at[pl.ds(sid * rpt, rpt)], sst)
  to = pltpu.async_copy(w_out.at[pl.ds(sid * rpt, rpt)],
                        spm_o.at[pl.ds(sid * rpt, rpt)], sst)
  ti.wait()
  to.wait()
  plsc.subcore_barrier()

  bufs = ((ba0, bb0, sa0, sb0), (ba1, bb1, sa1, sb1))

  def run(n_chunks, chunk0):
    def issue(j, slot):
      ba, bb, sa, sb = bufs[slot]
      ia = idx_a.at[pl.ds(j * CHUNK, CHUNK)]
      pltpu.async_copy(spm_i.at[ia], ba, sa)

    def finish(j, slot):
      ba, bb, sa, sb = bufs[slot]
      ia = idx_a.at[pl.ds(j * CHUNK, CHUNK)]
      ib = idx_b.at[pl.ds(j * CHUNK, CHUNK)]
      pltpu.make_async_copy(spm_i.at[ia], ba, sa).wait()
      pltpu.async_copy(spm_o.at[ib], ba, sb, add=True).wait()

      g = chunk0 + j

      @pl.when(g < N_CHUNKS - 1)
      def _():
        pltpu.sync_copy(ba, out.at[pl.ds(g * CHUNK, CHUNK)])

      @pl.when(g == N_CHUNKS - 1)
      def _():
        pltpu.sync_copy(ba.at[pl.ds(0, TAIL)],
                        out.at[pl.ds(g * CHUNK, TAIL)])

    issue(0, 0)

    def pair_body(p, carry):
      for s in range(2):
        j = 2 * p + s

        @pl.when(j + 1 < n_chunks)
        def _():
          issue(j + 1, 1 - s)

        finish(j, s)
      return carry

    lax.fori_loop(0, n_chunks // 2, pair_body, 0)
    if n_chunks % 2:
      finish(n_chunks - 1, (n_chunks - 1) % 2)

  @pl.when(wid < BIG_W)
  def _():
    n = KBIG * CHUNK
    c0 = wid * KBIG
    pltpu.sync_copy(in_idx.at[pl.ds(c0 * CHUNK, n)], idx_a.at[pl.ds(0, n)])
    pltpu.sync_copy(out_idx.at[pl.ds(c0 * CHUNK, n)], idx_b.at[pl.ds(0, n)])
    run(KBIG, c0)

  @pl.when((wid >= BIG_W) & (wid < 31))
  def _():
    n = KSML * CHUNK
    c0 = BIG_W * KBIG + (wid - BIG_W) * KSML
    pltpu.sync_copy(in_idx.at[pl.ds(c0 * CHUNK, n)], idx_a.at[pl.ds(0, n)])
    pltpu.sync_copy(out_idx.at[pl.ds(c0 * CHUNK, n)], idx_b.at[pl.ds(0, n)])
    run(KSML, c0)

  @pl.when(wid == 31)
  def _():
    pltpu.sync_copy(in_idx.at[pl.ds(LAST_W_ROW0, LAST_FULL)],
                    idx_a.at[pl.ds(0, LAST_FULL)])
    pltpu.sync_copy(out_idx.at[pl.ds(LAST_W_ROW0, LAST_FULL)],
                    idx_b.at[pl.ds(0, LAST_FULL)])
    pltpu.sync_copy(tin, idx_a.at[pl.ds(LAST_FULL, CHUNK)])
    pltpu.sync_copy(tout, idx_b.at[pl.ds(LAST_FULL, CHUNK)])
    run(KSML, LAST_W_ROW0 // CHUNK)


@jax.jit
def kernel(in_deg, out_deg, W_in, W_out):
  in_p = in_deg.astype(jnp.int32)
  out_p = out_deg.astype(jnp.int32)
  tin = jnp.pad(in_p[(N_CHUNKS - 1) * CHUNK:], (0, CHUNK - TAIL))
  tout = jnp.pad(out_p[(N_CHUNKS - 1) * CHUNK:], (0, CHUNK - TAIL))

  mesh = plsc.VectorSubcoreMesh(core_axis_name="c", subcore_axis_name="s")
  f = pl.kernel(
      _body,
      out_type=jax.ShapeDtypeStruct((N_NODES, HIDDEN), jnp.float32),
      mesh=mesh,
      scratch_types=[
          pltpu.VMEM((KBIG * CHUNK,), jnp.int32),
          pltpu.VMEM((KBIG * CHUNK,), jnp.int32),
          pltpu.VMEM((CHUNK, HIDDEN), jnp.float32),
          pltpu.VMEM((CHUNK, HIDDEN), jnp.float32),
          pltpu.VMEM((CHUNK, HIDDEN), jnp.float32),
          pltpu.VMEM((CHUNK, HIDDEN), jnp.float32),
          pltpu.VMEM_SHARED((VOCAB, HIDDEN), jnp.float32),
          pltpu.VMEM_SHARED((VOCAB, HIDDEN), jnp.float32),
          pltpu.SemaphoreType.DMA,
          pltpu.SemaphoreType.DMA,
          pltpu.SemaphoreType.DMA,
          pltpu.SemaphoreType.DMA,
          pltpu.SemaphoreType.DMA,
      ],
  )
  return f(in_p, out_p, tin, tout, W_in, W_out)


# speedup vs baseline: 1.1439x; 1.1230x over previous
import jax
import jax.numpy as jnp
from jax import lax
from jax.experimental import pallas as pl
from jax.experimental.pallas import tpu as pltpu
from jax.experimental.pallas import tpu_sc as plsc

N_NODES = 100000
HIDDEN = 128
CHUNK = 128
N_CHUNKS = (N_NODES + CHUNK - 1) // CHUNK
TAIL = N_NODES - (N_CHUNKS - 1) * CHUNK
BIG_W = N_CHUNKS - 24 * 32
KBIG, KSML = 25, 24
VOCAB = 512
LAST_W_ROW0 = (BIG_W * KBIG + (31 - BIG_W) * KSML) * CHUNK
LAST_FULL = (KSML - 1) * CHUNK


def _body(in_idx, out_idx, tin, tout, w_in, w_out, out, idx_a, idx_b, ba0,
          ba1, ba2, spm_i, spm_o, sa0, sa1, sa2, sb0, sb1, sb2, sw0, sw1, sw2,
          sst):
  cid = lax.axis_index("c")
  sid = lax.axis_index("s")
  wid = sid * 2 + cid

  rpt = VOCAB // 16
  ti = pltpu.async_copy(w_in.at[pl.ds(sid * rpt, rpt)],
                        spm_i.at[pl.ds(sid * rpt, rpt)], sst)
  to = pltpu.async_copy(w_out.at[pl.ds(sid * rpt, rpt)],
                        spm_o.at[pl.ds(sid * rpt, rpt)], sst)
  ti.wait()
  to.wait()
  plsc.subcore_barrier()

  ba = (ba0, ba1, ba2)
  sa = (sa0, sa1, sa2)
  sb = (sb0, sb1, sb2)
  sw = (sw0, sw1, sw2)

  def run(n, c0):
    def ia(j):
      return idx_a.at[pl.ds(j * CHUNK, CHUNK)]

    def ib(j):
      return idx_b.at[pl.ds(j * CHUNK, CHUNK)]

    def issue_a(j, m):
      pltpu.async_copy(spm_i.at[ia(j)], ba[m], sa[m])

    def wait_a(j, m):
      pltpu.make_async_copy(spm_i.at[ia(j)], ba[m], sa[m]).wait()

    def issue_b(j, m):
      pltpu.async_copy(spm_o.at[ib(j)], ba[m], sb[m], add=True)

    def wait_b(j, m):
      pltpu.make_async_copy(spm_o.at[ib(j)], ba[m], sb[m]).wait()

    def issue_w(j, m):
      g = c0 + j

      @pl.when(g < N_CHUNKS - 1)
      def _():
        pltpu.async_copy(ba[m], out.at[pl.ds(g * CHUNK, CHUNK)], sw[m])

      @pl.when(g == N_CHUNKS - 1)
      def _():
        pltpu.async_copy(ba[m].at[pl.ds(0, TAIL)],
                         out.at[pl.ds(g * CHUNK, TAIL)], sw[m])

    def wait_w(j, m):
      g = c0 + j

      @pl.when(g < N_CHUNKS - 1)
      def _():
        pltpu.make_async_copy(ba[m], out.at[pl.ds(g * CHUNK, CHUNK)],
                              sw[m]).wait()

      @pl.when(g == N_CHUNKS - 1)
      def _():
        pltpu.make_async_copy(ba[m].at[pl.ds(0, TAIL)],
                              out.at[pl.ds(g * CHUNK, TAIL)], sw[m]).wait()

    issue_a(0, 0)

    def triple_body(p, carry):
      for u in range(3):
        j = 3 * p + u

        @pl.when(j < n)
        def _():
          @pl.when(j >= 1)
          def _():
            wait_b(j - 1, (u - 1) % 3)
            issue_w(j - 1, (u - 1) % 3)

          @pl.when(j >= 2)
          def _():
            wait_w(j - 2, (u - 2) % 3)

          wait_a(j, u)
          issue_b(j, u)

          @pl.when(j + 1 < n)
          def _():
            issue_a(j + 1, (u + 1) % 3)

      return carry

    lax.fori_loop(0, (n + 2) // 3, triple_body, 0)

    m_last = (n - 1) % 3
    wait_b(n - 1, m_last)
    issue_w(n - 1, m_last)
    wait_w(n - 2, (n - 2) % 3)
    wait_w(n - 1, m_last)

  @pl.when(wid < BIG_W)
  def _():
    nr = KBIG * CHUNK
    c0 = wid * KBIG
    pltpu.sync_copy(in_idx.at[pl.ds(c0 * CHUNK, nr)], idx_a.at[pl.ds(0, nr)])
    pltpu.sync_copy(out_idx.at[pl.ds(c0 * CHUNK, nr)], idx_b.at[pl.ds(0, nr)])
    run(KBIG, c0)

  @pl.when((wid >= BIG_W) & (wid < 31))
  def _():
    nr = KSML * CHUNK
    c0 = BIG_W * KBIG + (wid - BIG_W) * KSML
    pltpu.sync_copy(in_idx.at[pl.ds(c0 * CHUNK, nr)], idx_a.at[pl.ds(0, nr)])
    pltpu.sync_copy(out_idx.at[pl.ds(c0 * CHUNK, nr)], idx_b.at[pl.ds(0, nr)])
    run(KSML, c0)

  @pl.when(wid == 31)
  def _():
    pltpu.sync_copy(in_idx.at[pl.ds(LAST_W_ROW0, LAST_FULL)],
                    idx_a.at[pl.ds(0, LAST_FULL)])
    pltpu.sync_copy(out_idx.at[pl.ds(LAST_W_ROW0, LAST_FULL)],
                    idx_b.at[pl.ds(0, LAST_FULL)])
    pltpu.sync_copy(tin, idx_a.at[pl.ds(LAST_FULL, CHUNK)])
    pltpu.sync_copy(tout, idx_b.at[pl.ds(LAST_FULL, CHUNK)])
    run(KSML, LAST_W_ROW0 // CHUNK)


@jax.jit
def kernel(in_deg, out_deg, W_in, W_out):
  in_p = in_deg.astype(jnp.int32)
  out_p = out_deg.astype(jnp.int32)
  tin = jnp.pad(in_p[(N_CHUNKS - 1) * CHUNK:], (0, CHUNK - TAIL))
  tout = jnp.pad(out_p[(N_CHUNKS - 1) * CHUNK:], (0, CHUNK - TAIL))

  mesh = plsc.VectorSubcoreMesh(core_axis_name="c", subcore_axis_name="s")
  f = pl.kernel(
      _body,
      out_type=jax.ShapeDtypeStruct((N_NODES, HIDDEN), jnp.float32),
      mesh=mesh,
      scratch_types=[
          pltpu.VMEM((KBIG * CHUNK,), jnp.int32),
          pltpu.VMEM((KBIG * CHUNK,), jnp.int32),
          pltpu.VMEM((CHUNK, HIDDEN), jnp.float32),
          pltpu.VMEM((CHUNK, HIDDEN), jnp.float32),
          pltpu.VMEM((CHUNK, HIDDEN), jnp.float32),
          pltpu.VMEM_SHARED((VOCAB, HIDDEN), jnp.float32),
          pltpu.VMEM_SHARED((VOCAB, HIDDEN), jnp.float32),
          pltpu.SemaphoreType.DMA,
          pltpu.SemaphoreType.DMA,
          pltpu.SemaphoreType.DMA,
          pltpu.SemaphoreType.DMA,
          pltpu.SemaphoreType.DMA,
          pltpu.SemaphoreType.DMA,
          pltpu.SemaphoreType.DMA,
          pltpu.SemaphoreType.DMA,
          pltpu.SemaphoreType.DMA,
          pltpu.SemaphoreType.DMA,
      ],
  )
  return f(in_p, out_p, tin, tout, W_in, W_out)
